# baseline (speedup 1.0000x reference)
import functools

import jax
import jax.numpy as jnp
import numpy as np
from jax import lax
from jax.experimental import pallas as pl
from jax.experimental.pallas import tpu as pltpu
from jax.experimental.pallas import tpu_sc as plsc

NC = 2
NS = 16
L = 16


def _pack_perm(D):
    Q = np.empty(D, np.int32)
    H = D // 2
    for k in range(H):
        Q[k] = 32 * (k // 16) + (k % 16)
        Q[H + k] = Q[k] + 16
    return Q


def _edge_embed_body(ea_ref, w_ref, b_ref, o_ref):
    o_ref[...] = (
        lax.dot_general(ea_ref[...], w_ref[...],
                        (((0,), (0,)), ((), ())),
                        preferred_element_type=jnp.float32)
        + b_ref[...]
    )


def _edge_embed(edge_attr, edge_lin_W, edge_lin_b):
    E, DE = edge_attr.shape
    D = edge_lin_W.shape[1]
    BE = 3200
    grid = E // BE
    return pl.pallas_call(
        _edge_embed_body,
        grid=(grid,),
        in_specs=[
            pl.BlockSpec((DE, BE), lambda i: (0, i)),
            pl.BlockSpec((DE, D), lambda i: (0, 0)),
            pl.BlockSpec((1, D), lambda i: (0, 0)),
        ],
        out_specs=pl.BlockSpec((BE, D), lambda i: (i, 0)),
        out_shape=jax.ShapeDtypeStruct((E, D), jnp.float32),
    )(edge_attr.T, edge_lin_W, edge_lin_b.reshape(1, D))


def _make_sc_agg(N, E, D, B, R):
    EPT = E // (NC * NS)
    CT = EPT // B
    KI = (CT - 2) // R
    ZR = B
    RPT = -(-N // NS // ZR) * ZR
    NP = RPT * NS
    assert EPT * NC * NS == E and KI * R + 2 == CT
    assert B % 8 == 0 and B <= 128 and RPT % ZR == 0 and R >= 3

    mesh = plsc.VectorSubcoreMesh(core_axis_name="c", subcore_axis_name="s")

    sc_types = (
        [pltpu.VMEM((B,), jnp.int32) for _ in range(R)]
        + [pltpu.VMEM((B,), jnp.int32) for _ in range(R)]
        + [pltpu.VMEM((B, D), jnp.float32) for _ in range(R)]
        + [pltpu.VMEM((B, D), jnp.float32) for _ in range(R)]
        + [pltpu.VMEM_SHARED((NP, D), jnp.float32)]
        + [pltpu.SemaphoreType.DMA for _ in range(5 * R)]
    )

    @functools.partial(
        pl.kernel,
        out_type=jax.ShapeDtypeStruct((NC, NP, D), jnp.float32),
        mesh=mesh,
        scratch_types=sc_types,
        compiler_params=pltpu.CompilerParams(needs_layout_passes=False),
    )
    def sc_agg(x_hbm, e_hbm, src_hbm, dst_hbm, out_hbm, *refs):
        src_c = refs[0 * R:1 * R]
        dst_c = refs[1 * R:2 * R]
        e_c = refs[2 * R:3 * R]
        x_c = refs[3 * R:4 * R]
        agg_sh = refs[4 * R]
        sem_i = refs[4 * R + 1:4 * R + 1 + R]
        sem_d = refs[4 * R + 1 + R:4 * R + 1 + 2 * R]
        sem_e = refs[4 * R + 1 + 2 * R:4 * R + 1 + 3 * R]
        sem_g = refs[4 * R + 1 + 3 * R:4 * R + 1 + 4 * R]
        sem_s = refs[4 * R + 1 + 4 * R:4 * R + 1 + 5 * R]
        c = lax.axis_index("c")
        s = lax.axis_index("s")
        wid = c * NS + s
        ebase = wid * EPT

        zero = jnp.zeros((L,), jnp.float32)

        def zrow(i, _):
            for j in range(D // L):
                x_c[0][i, pl.ds(j * L, L)] = zero
            return 0

        lax.fori_loop(0, ZR, zrow, 0)
        for k in range(RPT // ZR):
            pltpu.sync_copy(x_c[0], agg_sh.at[pl.ds(s * RPT + k * ZR, ZR)])
        plsc.subcore_barrier()

        def fetch_src(ci, u):
            base = ebase + ci * B
            pltpu.async_copy(src_hbm.at[pl.ds(base, B)], src_c[u], sem_i[u])

        def wait_src(u):
            pltpu.make_async_copy(src_hbm.at[pl.ds(0, B)], src_c[u],
                                  sem_i[u]).wait()

        def fetch_dst(ci, u):
            base = ebase + ci * B
            pltpu.async_copy(dst_hbm.at[pl.ds(base, B)], dst_c[u], sem_d[u])

        def wait_dst(u):
            pltpu.make_async_copy(dst_hbm.at[pl.ds(0, B)], dst_c[u],
                                  sem_d[u]).wait()

        def fetch_e(ci, u):
            base = ebase + ci * B
            pltpu.async_copy(e_hbm.at[pl.ds(base, B)], e_c[u], sem_e[u])

        def wait_e(u):
            pltpu.make_async_copy(e_hbm.at[pl.ds(0, B)], e_c[u],
                                  sem_e[u]).wait()

        def fetch_x(u):
            pltpu.async_copy(x_hbm.at[src_c[u]], x_c[u], sem_g[u])

        def wait_x(u):
            pltpu.make_async_copy(x_hbm.at[src_c[u]], x_c[u],
                                  sem_g[u]).wait()

        def compute_scatter(u):
            def edge(r4, _):
                for rr in range(4):
                    r = r4 * 4 + rr
                    for g in range(D // L):
                        sl = pl.ds(L * g, L)
                        x_c[u][r, sl] = jnp.maximum(
                            x_c[u][r, sl] + e_c[u][r, sl], 0.0)
                return 0

            lax.fori_loop(0, B // 4, edge, 0)
            pltpu.async_copy(x_c[u], agg_sh.at[dst_c[u]], sem_s[u], add=True)

        def wait_scatter(u):
            pltpu.make_async_copy(x_c[u], agg_sh.at[dst_c[u]],
                                  sem_s[u]).wait()

        for u in range(2):
            fetch_src(u, u)
            fetch_dst(u, u)
            fetch_e(u, u)
        wait_src(0)
        fetch_x(0)

        def body(k, _):
            for u in range(R):
                q = (u + 2) % R
                w = (u + 1) % R
                fetch_src(R * k + u + 2, q)
                wait_src(w)
                fetch_x(w)
                wait_e(u)
                wait_x(u)
                wait_dst(u)
                compute_scatter(u)
                if u < R - 2:
                    @pl.when(k > 0)
                    def _():
                        wait_scatter(q)
                else:
                    wait_scatter(q)
                fetch_dst(R * k + u + 2, q)
                fetch_e(R * k + u + 2, q)
            return 0

        lax.fori_loop(0, KI, body, 0)
        wait_src(1)
        fetch_x(1)
        wait_e(0)
        wait_x(0)
        wait_dst(0)
        compute_scatter(0)
        wait_e(1)
        wait_x(1)
        wait_dst(1)
        compute_scatter(1)
        for u in range(R):
            wait_scatter(u)

        plsc.subcore_barrier()
        pltpu.sync_copy(agg_sh.at[pl.ds(s * RPT, RPT)],
                        out_hbm.at[c, pl.ds(s * RPT, RPT)])

    return sc_agg


def _mlp_bn_body(x_ref, agg_ref, w1_ref, b1_ref, w2_ref, b2_ref,
                 eps_ref, g_ref, bt_ref, o_ref):
    n = x_ref.shape[0]
    h = ((1.0 + eps_ref[0, 0]) * x_ref[...]
         + agg_ref[0, :n] + agg_ref[1, :n])
    h = jnp.maximum(
        jnp.dot(h, w1_ref[...], preferred_element_type=jnp.float32)
        + b1_ref[...], 0.0)
    h = (jnp.dot(h, w2_ref[...], preferred_element_type=jnp.float32)
         + b2_ref[...])
    mean = jnp.sum(h, axis=0, keepdims=True) * (1.0 / n)
    var = jnp.sum((h - mean) ** 2, axis=0, keepdims=True) * (1.0 / n)
    hn = (h - mean) * lax.rsqrt(var + 1e-5) * g_ref[...] + bt_ref[...]
    o_ref[...] = jnp.maximum(hn, 0.0)


def _mlp_bn(x, agg, W1, b1, W2, b2, eps, gamma, beta, *, interpret=False):
    N, D = x.shape
    return pl.pallas_call(
        _mlp_bn_body,
        out_shape=jax.ShapeDtypeStruct((N, D), jnp.float32),
        interpret=interpret,
    )(x, agg, W1, b1.reshape(1, D), W2, b2.reshape(1, D),
      eps.reshape(1, 1), gamma.reshape(1, D), beta.reshape(1, D))


def kernel(x, edge_index, edge_attr, edge_lin_W, edge_lin_b,
           W1, b1, W2, b2, eps, gamma, beta):
    N, D = x.shape
    E = edge_index.shape[1]
    src = edge_index[0]
    dst = edge_index[1]
    e = _edge_embed(edge_attr, edge_lin_W, edge_lin_b)
    sc_agg = _make_sc_agg(N, E, D, B=40, R=4)
    agg = sc_agg(x, e, src, dst)
    return _mlp_bn(x, agg, W1, b1, W2, b2, eps, gamma, beta)


# speedup vs baseline: 1.0172x; 1.0172x over previous
import functools

import jax
import jax.numpy as jnp
import numpy as np
from jax import lax
from jax.experimental import pallas as pl
from jax.experimental.pallas import tpu as pltpu
from jax.experimental.pallas import tpu_sc as plsc

NC = 2
NS = 16
L = 16


def _pack_perm(D):
    Q = np.empty(D, np.int32)
    H = D // 2
    for k in range(H):
        Q[k] = 32 * (k // 16) + (k % 16)
        Q[H + k] = Q[k] + 16
    return Q


def _edge_embed_body(ea_ref, w_ref, b_ref, o_ref):
    o_ref[...] = (
        lax.dot_general(ea_ref[...], w_ref[...],
                        (((0,), (0,)), ((), ())),
                        preferred_element_type=jnp.float32)
        + b_ref[...]
    )


def _edge_embed(edge_attr_T, edge_lin_W, edge_lin_b):
    DE, E = edge_attr_T.shape
    D = edge_lin_W.shape[1]
    BE = 3200
    grid = E // BE
    return pl.pallas_call(
        _edge_embed_body,
        grid=(grid,),
        in_specs=[
            pl.BlockSpec((DE, BE), lambda i: (0, i)),
            pl.BlockSpec((DE, D), lambda i: (0, 0)),
            pl.BlockSpec((1, D), lambda i: (0, 0)),
        ],
        out_specs=pl.BlockSpec((BE, D), lambda i: (i, 0)),
        out_shape=jax.ShapeDtypeStruct((E, D), jnp.float32),
    )(edge_attr_T, edge_lin_W, edge_lin_b.reshape(1, D))


def _make_sc_agg(N, E, D, B, R):
    EPT = E // (NC * NS)
    CT = EPT // B
    KI = (CT - 2) // R
    ZR = B
    RPT = -(-N // NS // ZR) * ZR
    NP = RPT * NS
    assert EPT * NC * NS == E and KI * R + 2 == CT
    assert B % 8 == 0 and B <= 128 and RPT % ZR == 0 and R >= 3

    mesh = plsc.VectorSubcoreMesh(core_axis_name="c", subcore_axis_name="s")

    sc_types = (
        [pltpu.VMEM((B,), jnp.int32) for _ in range(R)]
        + [pltpu.VMEM((B,), jnp.int32) for _ in range(R)]
        + [pltpu.VMEM((B, D), jnp.float32) for _ in range(R)]
        + [pltpu.VMEM((B, D), jnp.float32) for _ in range(R)]
        + [pltpu.VMEM_SHARED((NP, D), jnp.float32)]
        + [pltpu.SemaphoreType.DMA for _ in range(5 * R)]
    )

    @functools.partial(
        pl.kernel,
        out_type=jax.ShapeDtypeStruct((NC, NP, D), jnp.float32),
        mesh=mesh,
        scratch_types=sc_types,
        compiler_params=pltpu.CompilerParams(needs_layout_passes=False),
    )
    def sc_agg(x_hbm, e_hbm, src_hbm, dst_hbm, out_hbm, *refs):
        src_c = refs[0 * R:1 * R]
        dst_c = refs[1 * R:2 * R]
        e_c = refs[2 * R:3 * R]
        x_c = refs[3 * R:4 * R]
        agg_sh = refs[4 * R]
        sem_i = refs[4 * R + 1:4 * R + 1 + R]
        sem_d = refs[4 * R + 1 + R:4 * R + 1 + 2 * R]
        sem_e = refs[4 * R + 1 + 2 * R:4 * R + 1 + 3 * R]
        sem_g = refs[4 * R + 1 + 3 * R:4 * R + 1 + 4 * R]
        sem_s = refs[4 * R + 1 + 4 * R:4 * R + 1 + 5 * R]
        c = lax.axis_index("c")
        s = lax.axis_index("s")
        wid = c * NS + s
        ebase = wid * EPT

        zero = jnp.zeros((L,), jnp.float32)

        def zrow(i, _):
            for j in range(D // L):
                x_c[0][i, pl.ds(j * L, L)] = zero
            return 0

        lax.fori_loop(0, ZR, zrow, 0)
        for k in range(RPT // ZR):
            pltpu.sync_copy(x_c[0], agg_sh.at[pl.ds(s * RPT + k * ZR, ZR)])
        plsc.subcore_barrier()

        def fetch_src(ci, u):
            base = ebase + ci * B
            pltpu.async_copy(src_hbm.at[pl.ds(base, B)], src_c[u], sem_i[u])

        def wait_src(u):
            pltpu.make_async_copy(src_hbm.at[pl.ds(0, B)], src_c[u],
                                  sem_i[u]).wait()

        def fetch_dst(ci, u):
            base = ebase + ci * B
            pltpu.async_copy(dst_hbm.at[pl.ds(base, B)], dst_c[u], sem_d[u])

        def wait_dst(u):
            pltpu.make_async_copy(dst_hbm.at[pl.ds(0, B)], dst_c[u],
                                  sem_d[u]).wait()

        def fetch_e(ci, u):
            base = ebase + ci * B
            pltpu.async_copy(e_hbm.at[pl.ds(base, B)], e_c[u], sem_e[u])

        def wait_e(u):
            pltpu.make_async_copy(e_hbm.at[pl.ds(0, B)], e_c[u],
                                  sem_e[u]).wait()

        def fetch_x(u):
            pltpu.async_copy(x_hbm.at[src_c[u]], x_c[u], sem_g[u])

        def wait_x(u):
            pltpu.make_async_copy(x_hbm.at[src_c[u]], x_c[u],
                                  sem_g[u]).wait()

        def compute_scatter(u):
            def edge(r4, _):
                for rr in range(4):
                    r = r4 * 4 + rr
                    for g in range(D // L):
                        sl = pl.ds(L * g, L)
                        x_c[u][r, sl] = jnp.maximum(
                            x_c[u][r, sl] + e_c[u][r, sl], 0.0)
                return 0

            lax.fori_loop(0, B // 4, edge, 0)
            pltpu.async_copy(x_c[u], agg_sh.at[dst_c[u]], sem_s[u], add=True)

        def wait_scatter(u):
            pltpu.make_async_copy(x_c[u], agg_sh.at[dst_c[u]],
                                  sem_s[u]).wait()

        for u in range(2):
            fetch_src(u, u)
            fetch_dst(u, u)
            fetch_e(u, u)
        wait_src(0)
        fetch_x(0)

        def body(k, _):
            for u in range(R):
                q = (u + 2) % R
                w = (u + 1) % R
                fetch_src(R * k + u + 2, q)
                wait_src(w)
                fetch_x(w)
                wait_e(u)
                wait_x(u)
                wait_dst(u)
                compute_scatter(u)
                if u < R - 2:
                    @pl.when(k > 0)
                    def _():
                        wait_scatter(q)
                else:
                    wait_scatter(q)
                fetch_dst(R * k + u + 2, q)
                fetch_e(R * k + u + 2, q)
            return 0

        lax.fori_loop(0, KI, body, 0)
        wait_src(1)
        fetch_x(1)
        wait_e(0)
        wait_x(0)
        wait_dst(0)
        compute_scatter(0)
        wait_e(1)
        wait_x(1)
        wait_dst(1)
        compute_scatter(1)
        for u in range(R):
            wait_scatter(u)

        plsc.subcore_barrier()
        pltpu.sync_copy(agg_sh.at[pl.ds(s * RPT, RPT)],
                        out_hbm.at[c, pl.ds(s * RPT, RPT)])

    return sc_agg


def _mlp_bn_body(x_ref, agg_ref, agg2_ref, w1_ref, b1_ref, w2_ref, b2_ref,
                 eps_ref, g_ref, bt_ref, o_ref):
    n = x_ref.shape[0]
    h = ((1.0 + eps_ref[0, 0]) * x_ref[...]
         + agg_ref[0, :n] + agg_ref[1, :n]
         + agg2_ref[0, :n] + agg2_ref[1, :n])
    h = jnp.maximum(
        jnp.dot(h, w1_ref[...], preferred_element_type=jnp.float32)
        + b1_ref[...], 0.0)
    h = (jnp.dot(h, w2_ref[...], preferred_element_type=jnp.float32)
         + b2_ref[...])
    mean = jnp.sum(h, axis=0, keepdims=True) * (1.0 / n)
    var = jnp.sum((h - mean) ** 2, axis=0, keepdims=True) * (1.0 / n)
    hn = (h - mean) * lax.rsqrt(var + 1e-5) * g_ref[...] + bt_ref[...]
    o_ref[...] = jnp.maximum(hn, 0.0)


def _mlp_bn(x, agg, agg2, W1, b1, W2, b2, eps, gamma, beta, *,
            interpret=False):
    N, D = x.shape
    return pl.pallas_call(
        _mlp_bn_body,
        out_shape=jax.ShapeDtypeStruct((N, D), jnp.float32),
        interpret=interpret,
    )(x, agg, agg2, W1, b1.reshape(1, D), W2, b2.reshape(1, D),
      eps.reshape(1, 1), gamma.reshape(1, D), beta.reshape(1, D))


def kernel(x, edge_index, edge_attr, edge_lin_W, edge_lin_b,
           W1, b1, W2, b2, eps, gamma, beta):
    N, D = x.shape
    E = edge_index.shape[1]
    EH = E // 2
    src = edge_index[0]
    dst = edge_index[1]
    ea_T = edge_attr.T
    sc_agg = _make_sc_agg(N, EH, D, B=40, R=3)
    e_a = _edge_embed(ea_T[:, :EH], edge_lin_W, edge_lin_b)
    agg_a = sc_agg(x, e_a, src[:EH], dst[:EH])
    e_b = _edge_embed(ea_T[:, EH:], edge_lin_W, edge_lin_b)
    agg_b = sc_agg(x, e_b, src[EH:], dst[EH:])
    return _mlp_bn(x, agg_a, agg_b, W1, b1, W2, b2, eps, gamma, beta)


# speedup vs baseline: 1.0549x; 1.0371x over previous
import functools

import jax
import jax.numpy as jnp
import numpy as np
from jax import lax
from jax.experimental import pallas as pl
from jax.experimental.pallas import tpu as pltpu
from jax.experimental.pallas import tpu_sc as plsc

NC = 2
NS = 16
L = 16


def _pack_perm(D):
    Q = np.empty(D, np.int32)
    H = D // 2
    for k in range(H):
        Q[k] = 32 * (k // 16) + (k % 16)
        Q[H + k] = Q[k] + 16
    return Q


def _edge_embed_body(ea_ref, w_ref, b_ref, o_ref):
    o_ref[...] = (
        lax.dot_general(ea_ref[...], w_ref[...],
                        (((0,), (0,)), ((), ())),
                        preferred_element_type=jnp.float32)
        + b_ref[...]
    )


def _edge_embed(edge_attr_T, edge_lin_W, edge_lin_b, ne, off):
    DE, E = edge_attr_T.shape
    D = edge_lin_W.shape[1]
    BE = 3200
    grid = ne // BE
    ob = off // BE
    return pl.pallas_call(
        _edge_embed_body,
        grid=(grid,),
        in_specs=[
            pl.BlockSpec((DE, BE), lambda i: (0, i + ob)),
            pl.BlockSpec((DE, D), lambda i: (0, 0)),
            pl.BlockSpec((1, D), lambda i: (0, 0)),
        ],
        out_specs=pl.BlockSpec((BE, D), lambda i: (i, 0)),
        out_shape=jax.ShapeDtypeStruct((ne, D), jnp.float32),
    )(edge_attr_T, edge_lin_W, edge_lin_b.reshape(1, D))


def _make_sc_agg(N, E, D, B, R, EOFF):
    EPT = E // (NC * NS)
    CT = EPT // B
    KI = (CT - 2) // R
    ZR = B
    RPT = -(-N // NS // ZR) * ZR
    NP = RPT * NS
    assert EPT * NC * NS == E and KI * R + 2 == CT
    assert B % 8 == 0 and B <= 128 and RPT % ZR == 0 and R >= 3

    mesh = plsc.VectorSubcoreMesh(core_axis_name="c", subcore_axis_name="s")

    sc_types = (
        [pltpu.VMEM((B,), jnp.int32) for _ in range(R)]
        + [pltpu.VMEM((B,), jnp.int32) for _ in range(R)]
        + [pltpu.VMEM((B, D), jnp.float32) for _ in range(R)]
        + [pltpu.VMEM((B, D), jnp.float32) for _ in range(R)]
        + [pltpu.VMEM_SHARED((NP, D), jnp.float32)]
        + [pltpu.SemaphoreType.DMA for _ in range(5 * R)]
    )

    @functools.partial(
        pl.kernel,
        out_type=jax.ShapeDtypeStruct((NC, NP, D), jnp.float32),
        mesh=mesh,
        scratch_types=sc_types,
        compiler_params=pltpu.CompilerParams(needs_layout_passes=False),
    )
    def sc_agg(x_hbm, e_hbm, src_hbm, dst_hbm, out_hbm, *refs):
        src_c = refs[0 * R:1 * R]
        dst_c = refs[1 * R:2 * R]
        e_c = refs[2 * R:3 * R]
        x_c = refs[3 * R:4 * R]
        agg_sh = refs[4 * R]
        sem_i = refs[4 * R + 1:4 * R + 1 + R]
        sem_d = refs[4 * R + 1 + R:4 * R + 1 + 2 * R]
        sem_e = refs[4 * R + 1 + 2 * R:4 * R + 1 + 3 * R]
        sem_g = refs[4 * R + 1 + 3 * R:4 * R + 1 + 4 * R]
        sem_s = refs[4 * R + 1 + 4 * R:4 * R + 1 + 5 * R]
        c = lax.axis_index("c")
        s = lax.axis_index("s")
        wid = c * NS + s
        ebase = wid * EPT
        ibase = EOFF + wid * EPT

        zero = jnp.zeros((L,), jnp.float32)

        def zrow(i, _):
            for j in range(D // L):
                x_c[0][i, pl.ds(j * L, L)] = zero
            return 0

        lax.fori_loop(0, ZR, zrow, 0)
        for k in range(RPT // ZR):
            pltpu.sync_copy(x_c[0], agg_sh.at[pl.ds(s * RPT + k * ZR, ZR)])
        plsc.subcore_barrier()

        def fetch_src(ci, u):
            base = ibase + ci * B
            pltpu.async_copy(src_hbm.at[pl.ds(base, B)], src_c[u], sem_i[u])

        def wait_src(u):
            pltpu.make_async_copy(src_hbm.at[pl.ds(0, B)], src_c[u],
                                  sem_i[u]).wait()

        def fetch_dst(ci, u):
            base = ibase + ci * B
            pltpu.async_copy(dst_hbm.at[pl.ds(base, B)], dst_c[u], sem_d[u])

        def wait_dst(u):
            pltpu.make_async_copy(dst_hbm.at[pl.ds(0, B)], dst_c[u],
                                  sem_d[u]).wait()

        def fetch_e(ci, u):
            base = ebase + ci * B
            pltpu.async_copy(e_hbm.at[pl.ds(base, B)], e_c[u], sem_e[u])

        def wait_e(u):
            pltpu.make_async_copy(e_hbm.at[pl.ds(0, B)], e_c[u],
                                  sem_e[u]).wait()

        def fetch_x(u):
            pltpu.async_copy(x_hbm.at[src_c[u]], x_c[u], sem_g[u])

        def wait_x(u):
            pltpu.make_async_copy(x_hbm.at[src_c[u]], x_c[u],
                                  sem_g[u]).wait()

        def compute_scatter(u):
            def edge(r4, _):
                for rr in range(4):
                    r = r4 * 4 + rr
                    for g in range(D // L):
                        sl = pl.ds(L * g, L)
                        x_c[u][r, sl] = jnp.maximum(
                            x_c[u][r, sl] + e_c[u][r, sl], 0.0)
                return 0

            lax.fori_loop(0, B // 4, edge, 0)
            pltpu.async_copy(x_c[u], agg_sh.at[dst_c[u]], sem_s[u], add=True)

        def wait_scatter(u):
            pltpu.make_async_copy(x_c[u], agg_sh.at[dst_c[u]],
                                  sem_s[u]).wait()

        for u in range(2):
            fetch_src(u, u)
            fetch_dst(u, u)
            fetch_e(u, u)
        wait_src(0)
        fetch_x(0)

        def body(k, _):
            for u in range(R):
                q = (u + 2) % R
                w = (u + 1) % R
                fetch_src(R * k + u + 2, q)
                wait_src(w)
                fetch_x(w)
                wait_e(u)
                wait_x(u)
                wait_dst(u)
                compute_scatter(u)
                if u < R - 2:
                    @pl.when(k > 0)
                    def _():
                        wait_scatter(q)
                else:
                    wait_scatter(q)
                fetch_dst(R * k + u + 2, q)
                fetch_e(R * k + u + 2, q)
            return 0

        lax.fori_loop(0, KI, body, 0)
        wait_src(1)
        fetch_x(1)
        wait_e(0)
        wait_x(0)
        wait_dst(0)
        compute_scatter(0)
        wait_e(1)
        wait_x(1)
        wait_dst(1)
        compute_scatter(1)
        for u in range(R):
            wait_scatter(u)

        plsc.subcore_barrier()
        pltpu.sync_copy(agg_sh.at[pl.ds(s * RPT, RPT)],
                        out_hbm.at[c, pl.ds(s * RPT, RPT)])

    return sc_agg


def _mlp_bn_body(x_ref, agg_ref, agg2_ref, w1_ref, b1_ref, w2_ref, b2_ref,
                 eps_ref, g_ref, bt_ref, o_ref):
    n = x_ref.shape[0]
    h = ((1.0 + eps_ref[0, 0]) * x_ref[...]
         + agg_ref[0, :n] + agg_ref[1, :n]
         + agg2_ref[0, :n] + agg2_ref[1, :n])
    h = jnp.maximum(
        jnp.dot(h, w1_ref[...], preferred_element_type=jnp.float32)
        + b1_ref[...], 0.0)
    h = (jnp.dot(h, w2_ref[...], preferred_element_type=jnp.float32)
         + b2_ref[...])
    mean = jnp.sum(h, axis=0, keepdims=True) * (1.0 / n)
    var = jnp.sum((h - mean) ** 2, axis=0, keepdims=True) * (1.0 / n)
    hn = (h - mean) * lax.rsqrt(var + 1e-5) * g_ref[...] + bt_ref[...]
    o_ref[...] = jnp.maximum(hn, 0.0)


def _mlp_bn(x, agg, agg2, W1, b1, W2, b2, eps, gamma, beta, *,
            interpret=False):
    N, D = x.shape
    return pl.pallas_call(
        _mlp_bn_body,
        out_shape=jax.ShapeDtypeStruct((N, D), jnp.float32),
        interpret=interpret,
    )(x, agg, agg2, W1, b1.reshape(1, D), W2, b2.reshape(1, D),
      eps.reshape(1, 1), gamma.reshape(1, D), beta.reshape(1, D))


def kernel(x, edge_index, edge_attr, edge_lin_W, edge_lin_b,
           W1, b1, W2, b2, eps, gamma, beta):
    N, D = x.shape
    E = edge_index.shape[1]
    EH = E // 2
    src = edge_index[0]
    dst = edge_index[1]
    ea_T = edge_attr.T
    sc_agg_a = _make_sc_agg(N, EH, D, B=40, R=3, EOFF=0)
    sc_agg_b = _make_sc_agg(N, EH, D, B=40, R=3, EOFF=EH)
    e_a = _edge_embed(ea_T, edge_lin_W, edge_lin_b, EH, 0)
    agg_a = sc_agg_a(x, e_a, src, dst)
    e_b = _edge_embed(ea_T, edge_lin_W, edge_lin_b, EH, EH)
    agg_b = sc_agg_b(x, e_b, src, dst)
    return _mlp_bn(x, agg_a, agg_b, W1, b1, W2, b2, eps, gamma, beta)
